# Initial kernel scaffold; baseline (speedup 1.0000x reference)
import jax
import jax.numpy as jnp
from jax.experimental import pallas as pl


def kernel(x, batchidx, condition, params):
    raise NotImplementedError("write your pallas kernel here")


# speedup vs baseline: 6.9732x; 6.9732x over previous
import functools
import math

import jax
import jax.numpy as jnp
import numpy as np
from jax.experimental import pallas as pl
from jax.experimental.pallas import tpu as pltpu

B = 64
PTS = 256
F0 = 64
E = 128
H = 4
DH = E // H
NODES = [16, 4]
NCOND = 6
SLOPE = 0.01
GPP = 8


def _lrelu(x):
    return jnp.where(x >= 0, x, SLOPE * x)


def _prep_ffn(p):
    return {'W1': p['W1'], 'b1': p['b1'][None, :],
            'W2': p['W2'], 'b2': p['b2'][None, :]}


def _prep_cnu(p):
    n_lat = p['emb']['W2'].shape[1]
    gW1 = p['glob']['W1']
    oW1 = p['out']['W1']
    return {
        'emb': _prep_ffn(p['emb']),
        'gc': gW1[0:1], 'gmean': gW1[1:1 + n_lat],
        'gmad': gW1[1 + n_lat:1 + 2 * n_lat],
        'gmax': gW1[1 + 2 * n_lat:1 + 3 * n_lat],
        'gb1': p['glob']['b1'][None, :], 'gW2': p['glob']['W2'],
        'gb2': p['glob']['b2'][None, :],
        'oxl': oW1[:n_lat], 'oxg': oW1[n_lat:],
        'ob1': p['out']['b1'][None, :], 'oW2': p['out']['W2'],
        'ob2': p['out']['b2'][None, :],
    }


def _prep_disc(p):
    d = p['layers'][0]['emb']['W1'].shape[0]
    dW1 = p['disc']['W1']
    return {
        'layers': [_prep_cnu(lp) for lp in p['layers']],
        'dc': dW1[0:1], 'dmean': dW1[1:1 + d], 'dmad': dW1[1 + d:1 + 2 * d],
        'dmax': dW1[1 + 2 * d:1 + 3 * d], 'dcond': dW1[1 + 3 * d:],
        'db1': p['disc']['b1'][None, :], 'dW2': p['disc']['W2'],
        'db2': p['disc']['b2'][None, :],
    }


def _prep_pool(p):
    return {
        'xcent': p['xcent_base'],
        'Wq': [p['Wq'][:, h * DH:(h + 1) * DH] for h in range(H)],
        'bq': [p['bq'][None, h * DH:(h + 1) * DH] for h in range(H)],
        'Wk': [p['Wk'][:, h * DH:(h + 1) * DH] for h in range(H)],
        'bk': [p['bk'][None, h * DH:(h + 1) * DH] for h in range(H)],
        'Wv': [p['Wv'][:, h * DH:(h + 1) * DH] for h in range(H)],
        'bv': [p['bv'][None, h * DH:(h + 1) * DH] for h in range(H)],
        'Wo': [p['Wo'][h * DH:(h + 1) * DH, :] for h in range(H)],
        'bo': p['bo'][None, :],
    }


def _mm(a, b):
    return jax.lax.dot_general(a, b, (((a.ndim - 1,), (0,)), ((), ())),
                               preferred_element_type=jnp.float32)


def _ffn(w, x, final_linear=False):
    h = _lrelu(_mm(x, w['W1']) + w['b1'])
    o = _mm(h, w['W2']) + w['b2']
    return o if final_linear else _lrelu(o)


def _gmp(x2d, r):
    d = x2d.shape[-1]
    x3 = x2d.reshape(GPP, r, d)
    mean = jnp.mean(x3, axis=1)
    mad = jnp.mean(jnp.abs(x3 - mean[:, None, :]), axis=1)
    mx = jnp.max(x3, axis=1)
    return mean, mad, mx


def _cnu(w, x2d, r):
    xl = _ffn(w['emb'], x2d)
    mean, mad, mx = _gmp(xl, r)
    gh = _lrelu(float(r) * w['gc'] + _mm(mean, w['gmean'])
                + _mm(mad, w['gmad']) + _mm(mx, w['gmax']) + w['gb1'])
    xg = _lrelu(_mm(gh, w['gW2']) + w['gb2'])
    t = _mm(xg, w['oxg'])
    hid = t.shape[-1]
    tb = jnp.broadcast_to(t[:, None, :], (GPP, r, hid)).reshape(GPP * r, hid)
    oh = _lrelu(_mm(xl, w['oxl']) + tb + w['ob1'])
    return _mm(oh, w['oW2']) + w['ob2']


def _disc(w, x2d, cond, r):
    for lw in w['layers']:
        x2d = x2d + _cnu(lw, x2d, r)
    mean, mad, mx = _gmp(x2d, r)
    h = _lrelu(float(r) * w['dc'] + _mm(mean, w['dmean'])
               + _mm(mad, w['dmad']) + _mm(mx, w['dmax'])
               + _mm(cond, w['dcond']) + w['db1'])
    return _mm(h, w['dW2']) + w['db2']


def _pool(w, x2d, s, r):
    scale = 1.0 / math.sqrt(DH)
    acc = None
    for h in range(H):
        q = _mm(w['xcent'], w['Wq'][h]) + w['bq'][h]
        k = _mm(x2d, w['Wk'][h]) + w['bk'][h]
        v = _mm(x2d, w['Wv'][h]) + w['bv'][h]
        u = _mm(v, w['Wo'][h])
        sc = jax.lax.dot_general(k, q, (((1,), (1,)), ((), ())),
                                 preferred_element_type=jnp.float32)
        sc = (sc * scale).reshape(GPP, s, r)
        m = jnp.max(sc, axis=1, keepdims=True)
        e = jnp.exp(sc - m)
        a = e / jnp.sum(e, axis=1, keepdims=True)
        u3 = u.reshape(GPP, s, E)
        oh = jax.lax.dot_general(a, u3, (((1,), (1,)), ((0,), (0,))),
                                 preferred_element_type=jnp.float32)
        acc = oh if acc is None else acc + oh
    return (acc + w['bo'][None]).reshape(GPP * r, E)


def _body(x_ref, cond_ref, w_ref, o0_ref, o1_ref, o2_ref):
    x = x_ref[...]
    cond = cond_ref[...]
    w = jax.tree.map(lambda ref: ref[...], w_ref,
                     is_leaf=lambda n: hasattr(n, 'dtype') and hasattr(n, 'at'))

    o0_ref[...] = _disc(w['disc'][0], x, cond, PTS)

    x0 = _ffn(w['emb'][0]['inp'], x, final_linear=True)
    xe = _cnu(w['emb'][0]['cnu'], x0, PTS) + x0
    x1 = _pool(w['pool'][0], xe, PTS, NODES[0])

    o1_ref[...] = _disc(w['disc'][1], x1, cond, NODES[0])

    xi = _ffn(w['emb'][1]['inp'], x1, final_linear=True)
    xe1 = _cnu(w['emb'][1]['cnu'], xi, NODES[0]) + xi
    x2 = _pool(w['pool'][1], xe1, NODES[0], NODES[1])

    o2_ref[...] = _disc(w['disc'][2], x2, cond, NODES[1])


def _full_spec(a):
    nd = a.ndim
    return pl.BlockSpec(a.shape, lambda i, _n=nd: (0,) * _n)


@jax.jit
def kernel(x, batchidx, condition, params):
    del batchidx
    w = {
        'disc': [_prep_disc(p) for p in params['disc']],
        'emb': [{'inp': _prep_ffn(p['inp']), 'cnu': _prep_cnu(p['cnu'])}
                for p in params['emb']],
        'pool': [_prep_pool(p) for p in params['pool']],
    }
    grid = B // GPP
    wspecs = jax.tree.map(_full_spec, w)
    s0, s1, s2 = pl.pallas_call(
        _body,
        grid=(grid,),
        in_specs=[
            pl.BlockSpec((GPP * PTS, F0), lambda i: (i, 0)),
            pl.BlockSpec((GPP, NCOND), lambda i: (i, 0)),
            wspecs,
        ],
        out_specs=[pl.BlockSpec((GPP, 1), lambda i: (i, 0))] * 3,
        out_shape=[jax.ShapeDtypeStruct((B, 1), jnp.float32)] * 3,
        compiler_params=pltpu.CompilerParams(
            dimension_semantics=("arbitrary",)),
    )(x, condition, w)
    return jnp.concatenate([s0, s1, s2], axis=0)


# speedup vs baseline: 9.5069x; 1.3634x over previous
import functools
import math

import jax
import jax.numpy as jnp
import numpy as np
from jax.experimental import pallas as pl
from jax.experimental.pallas import tpu as pltpu

B = 64
PTS = 256
F0 = 64
E = 128
H = 4
DH = E // H
NODES = [16, 4]
NCOND = 6
SLOPE = 0.01
GPP = 8


def _lrelu(x):
    return jnp.where(x >= 0, x, SLOPE * x)


def _prep_ffn(p):
    return {'W1': p['W1'], 'b1': p['b1'][None, :],
            'W2': p['W2'], 'b2': p['b2'][None, :]}


def _prep_cnu(p):
    n_lat = p['emb']['W2'].shape[1]
    gW1 = p['glob']['W1']
    oW1 = p['out']['W1']
    return {
        'emb': _prep_ffn(p['emb']),
        'gc': gW1[0:1], 'gmean': gW1[1:1 + n_lat],
        'gmad': gW1[1 + n_lat:1 + 2 * n_lat],
        'gmax': gW1[1 + 2 * n_lat:1 + 3 * n_lat],
        'gb1': p['glob']['b1'][None, :], 'gW2': p['glob']['W2'],
        'gb2': p['glob']['b2'][None, :],
        'oxl': oW1[:n_lat], 'oxg': oW1[n_lat:],
        'ob1': p['out']['b1'][None, :], 'oW2': p['out']['W2'],
        'ob2': p['out']['b2'][None, :],
    }


def _prep_disc(p):
    d = p['layers'][0]['emb']['W1'].shape[0]
    dW1 = p['disc']['W1']
    return {
        'layers': [_prep_cnu(lp) for lp in p['layers']],
        'dc': dW1[0:1], 'dmean': dW1[1:1 + d], 'dmad': dW1[1 + d:1 + 2 * d],
        'dmax': dW1[1 + 2 * d:1 + 3 * d], 'dcond': dW1[1 + 3 * d:],
        'db1': p['disc']['b1'][None, :], 'dW2': p['disc']['W2'],
        'db2': p['disc']['b2'][None, :],
    }


def _prep_pool(p):
    return {
        'xcent': p['xcent_base'],
        'Wq': [p['Wq'][:, h * DH:(h + 1) * DH] for h in range(H)],
        'bq': [p['bq'][None, h * DH:(h + 1) * DH] for h in range(H)],
        'Wk': [p['Wk'][:, h * DH:(h + 1) * DH] for h in range(H)],
        'bk': [p['bk'][None, h * DH:(h + 1) * DH] for h in range(H)],
        'Wv': [p['Wv'][:, h * DH:(h + 1) * DH] for h in range(H)],
        'bv': [p['bv'][None, h * DH:(h + 1) * DH] for h in range(H)],
        'Wo': [p['Wo'][h * DH:(h + 1) * DH, :] for h in range(H)],
        'bo': p['bo'][None, :],
    }


def _mm(a, b):
    return jax.lax.dot_general(a, b, (((a.ndim - 1,), (0,)), ((), ())),
                               preferred_element_type=jnp.float32)


def _ffn(w, x, final_linear=False):
    h = _lrelu(_mm(x, w['W1']) + w['b1'])
    o = _mm(h, w['W2']) + w['b2']
    return o if final_linear else _lrelu(o)


def _gmp(x2d, r):
    d = x2d.shape[-1]
    x3 = x2d.reshape(GPP, r, d)
    mean = jnp.mean(x3, axis=1)
    mad = jnp.mean(jnp.abs(x3 - mean[:, None, :]), axis=1)
    mx = jnp.max(x3, axis=1)
    return mean, mad, mx


def _cnu(w, x2d, r):
    xl = _ffn(w['emb'], x2d)
    mean, mad, mx = _gmp(xl, r)
    gh = _lrelu(float(r) * w['gc'] + _mm(mean, w['gmean'])
                + _mm(mad, w['gmad']) + _mm(mx, w['gmax']) + w['gb1'])
    xg = _lrelu(_mm(gh, w['gW2']) + w['gb2'])
    t = _mm(xg, w['oxg'])
    hid = t.shape[-1]
    tb = jnp.broadcast_to(t[:, None, :], (GPP, r, hid)).reshape(GPP * r, hid)
    oh = _lrelu(_mm(xl, w['oxl']) + tb + w['ob1'])
    return _mm(oh, w['oW2']) + w['ob2']


def _disc(w, x2d, cond, r):
    for lw in w['layers']:
        x2d = x2d + _cnu(lw, x2d, r)
    mean, mad, mx = _gmp(x2d, r)
    h = _lrelu(float(r) * w['dc'] + _mm(mean, w['dmean'])
               + _mm(mad, w['dmad']) + _mm(mx, w['dmax'])
               + _mm(cond, w['dcond']) + w['db1'])
    return _mm(h, w['dW2']) + w['db2']


def _pool(w, x2d, s, r):
    scale = 1.0 / math.sqrt(DH)
    acc = None
    for h in range(H):
        q = _mm(w['xcent'], w['Wq'][h]) + w['bq'][h]
        k = _mm(x2d, w['Wk'][h]) + w['bk'][h]
        v = _mm(x2d, w['Wv'][h]) + w['bv'][h]
        u = _mm(v, w['Wo'][h])
        sc = jax.lax.dot_general(k, q, (((1,), (1,)), ((), ())),
                                 preferred_element_type=jnp.float32)
        sc = (sc * scale).reshape(GPP, s, r)
        m = jnp.max(sc, axis=1, keepdims=True)
        e = jnp.exp(sc - m)
        a = e / jnp.sum(e, axis=1, keepdims=True)
        u3 = u.reshape(GPP, s, E)
        oh = jax.lax.dot_general(a, u3, (((1,), (1,)), ((0,), (0,))),
                                 preferred_element_type=jnp.float32)
        acc = oh if acc is None else acc + oh
    return (acc + w['bo'][None]).reshape(GPP * r, E)


def _body(x_ref, cond_ref, p_ref, o0_ref, o1_ref, o2_ref):
    x = x_ref[...]
    cond = cond_ref[...]
    p = jax.tree.map(lambda ref: ref[...], p_ref,
                     is_leaf=lambda n: hasattr(n, 'dtype') and hasattr(n, 'at'))
    w = {
        'disc': [_prep_disc(q) for q in p['disc']],
        'emb': [{'inp': _prep_ffn(q['inp']), 'cnu': _prep_cnu(q['cnu'])}
                for q in p['emb']],
        'pool': [_prep_pool(q) for q in p['pool']],
    }

    o0_ref[...] = _disc(w['disc'][0], x, cond, PTS)

    x0 = _ffn(w['emb'][0]['inp'], x, final_linear=True)
    xe = _cnu(w['emb'][0]['cnu'], x0, PTS) + x0
    x1 = _pool(w['pool'][0], xe, PTS, NODES[0])

    o1_ref[...] = _disc(w['disc'][1], x1, cond, NODES[0])

    xi = _ffn(w['emb'][1]['inp'], x1, final_linear=True)
    xe1 = _cnu(w['emb'][1]['cnu'], xi, NODES[0]) + xi
    x2 = _pool(w['pool'][1], xe1, NODES[0], NODES[1])

    o2_ref[...] = _disc(w['disc'][2], x2, cond, NODES[1])


def _full_spec(a):
    nd = a.ndim
    return pl.BlockSpec(a.shape, lambda i, _n=nd: (0,) * _n)


@jax.jit
def kernel(x, batchidx, condition, params):
    del batchidx
    grid = B // GPP
    wspecs = jax.tree.map(_full_spec, params)
    s0, s1, s2 = pl.pallas_call(
        _body,
        grid=(grid,),
        in_specs=[
            pl.BlockSpec((GPP * PTS, F0), lambda i: (i, 0)),
            pl.BlockSpec((GPP, NCOND), lambda i: (i, 0)),
            wspecs,
        ],
        out_specs=[pl.BlockSpec((GPP, 1), lambda i: (i, 0))] * 3,
        out_shape=[jax.ShapeDtypeStruct((B, 1), jnp.float32)] * 3,
        compiler_params=pltpu.CompilerParams(
            dimension_semantics=("arbitrary",)),
    )(x, condition, params)
    return jnp.concatenate([s0, s1, s2], axis=0)


# speedup vs baseline: 9.5427x; 1.0038x over previous
import functools
import math

import jax
import jax.numpy as jnp
import numpy as np
from jax.experimental import pallas as pl
from jax.experimental.pallas import tpu as pltpu

B = 64
PTS = 256
F0 = 64
E = 128
H = 4
DH = E // H
NODES = [16, 4]
NCOND = 6
SLOPE = 0.01
GPP = 8


def _lrelu(x):
    return jnp.where(x >= 0, x, SLOPE * x)


def _prep_ffn(p):
    return {'W1': p['W1'], 'b1': p['b1'][None, :],
            'W2': p['W2'], 'b2': p['b2'][None, :]}


def _prep_cnu(p):
    return {'emb': _prep_ffn(p['emb']), 'glob': _prep_ffn(p['glob']),
            'out': _prep_ffn(p['out'])}


def _prep_disc(p):
    return {'layers': [_prep_cnu(lp) for lp in p['layers']],
            'disc': _prep_ffn(p['disc'])}


def _prep_pool(p):
    return {'xcent': p['xcent_base'],
            'Wq': p['Wq'], 'bq': p['bq'][None, :],
            'Wk': p['Wk'], 'bk': p['bk'][None, :],
            'Wv': p['Wv'], 'bv': p['bv'][None, :],
            'Wo': p['Wo'], 'bo': p['bo'][None, :]}


def _mm(a, b):
    return jax.lax.dot_general(a, b, (((a.ndim - 1,), (0,)), ((), ())),
                               preferred_element_type=jnp.float32)


def _ffn(w, x, final_linear=False):
    h = _lrelu(_mm(x, w['W1']) + w['b1'])
    o = _mm(h, w['W2']) + w['b2']
    return o if final_linear else _lrelu(o)


def _gmp(x2d, r):
    d = x2d.shape[-1]
    x3 = x2d.reshape(GPP, r, d)
    mean = jnp.mean(x3, axis=1)
    mad = jnp.mean(jnp.abs(x3 - mean[:, None, :]), axis=1)
    mx = jnp.max(x3, axis=1)
    cnt = jnp.full((GPP, 1), float(r), jnp.float32)
    return cnt, mean, mad, mx


def _cnu(w, x2d, r):
    xl = _ffn(w['emb'], x2d)
    cnt, mean, mad, mx = _gmp(xl, r)
    g = jnp.concatenate([cnt, mean, mad, mx], axis=-1)
    xg = _ffn(w['glob'], g)
    n_glob = xg.shape[-1]
    xgb = jnp.broadcast_to(xg[:, None, :],
                           (GPP, r, n_glob)).reshape(GPP * r, n_glob)
    cat = jnp.concatenate([xl, xgb], axis=-1)
    return _ffn(w['out'], cat, final_linear=True)


def _disc(w, x2d, cond, r):
    for lw in w['layers']:
        x2d = x2d + _cnu(lw, x2d, r)
    cnt, mean, mad, mx = _gmp(x2d, r)
    inp = jnp.concatenate([cnt, mean, mad, mx, cond], axis=-1)
    return _ffn(w['disc'], inp, final_linear=True)


def _pool(w, x2d, s, r):
    scale = math.sqrt(DH)
    q = _mm(w['xcent'], w['Wq']) + w['bq']
    k = _mm(x2d, w['Wk']) + w['bk']
    v = _mm(x2d, w['Wv']) + w['bv']
    heads = []
    for h in range(H):
        sl = slice(h * DH, (h + 1) * DH)
        sc = jax.lax.dot_general(k[:, sl], q[:, sl], (((1,), (1,)), ((), ())),
                                 preferred_element_type=jnp.float32)
        sc = (sc / scale).reshape(GPP, s, r)
        m = jnp.max(sc, axis=1, keepdims=True)
        e = jnp.exp(sc - m)
        a = e / jnp.sum(e, axis=1, keepdims=True)
        v3 = v[:, sl].reshape(GPP, s, DH)
        heads.append(jax.lax.dot_general(
            a, v3, (((1,), (1,)), ((0,), (0,))),
            preferred_element_type=jnp.float32))
    o = jnp.concatenate(heads, axis=-1).reshape(GPP * r, E)
    return _mm(o, w['Wo']) + w['bo']


def _body(x_ref, cond_ref, p_ref, o0_ref, o1_ref, o2_ref):
    x = x_ref[...]
    cond = cond_ref[...]
    p = jax.tree.map(lambda ref: ref[...], p_ref,
                     is_leaf=lambda n: hasattr(n, 'dtype') and hasattr(n, 'at'))
    w = {
        'disc': [_prep_disc(q) for q in p['disc']],
        'emb': [{'inp': _prep_ffn(q['inp']), 'cnu': _prep_cnu(q['cnu'])}
                for q in p['emb']],
        'pool': [_prep_pool(q) for q in p['pool']],
    }

    o0_ref[...] = _disc(w['disc'][0], x, cond, PTS)

    x0 = _ffn(w['emb'][0]['inp'], x, final_linear=True)
    xe = _cnu(w['emb'][0]['cnu'], x0, PTS) + x0
    x1 = _pool(w['pool'][0], xe, PTS, NODES[0])

    o1_ref[...] = _disc(w['disc'][1], x1, cond, NODES[0])

    xi = _ffn(w['emb'][1]['inp'], x1, final_linear=True)
    xe1 = _cnu(w['emb'][1]['cnu'], xi, NODES[0]) + xi
    x2 = _pool(w['pool'][1], xe1, NODES[0], NODES[1])

    o2_ref[...] = _disc(w['disc'][2], x2, cond, NODES[1])


def _full_spec(a):
    nd = a.ndim
    return pl.BlockSpec(a.shape, lambda i, _n=nd: (0,) * _n)


@jax.jit
def kernel(x, batchidx, condition, params):
    del batchidx
    grid = B // GPP
    wspecs = jax.tree.map(_full_spec, params)
    s0, s1, s2 = pl.pallas_call(
        _body,
        grid=(grid,),
        in_specs=[
            pl.BlockSpec((GPP * PTS, F0), lambda i: (i, 0)),
            pl.BlockSpec((GPP, NCOND), lambda i: (i, 0)),
            wspecs,
        ],
        out_specs=[pl.BlockSpec((GPP, 1), lambda i: (i, 0))] * 3,
        out_shape=[jax.ShapeDtypeStruct((B, 1), jnp.float32)] * 3,
        compiler_params=pltpu.CompilerParams(
            dimension_semantics=("arbitrary",)),
    )(x, condition, params)
    return jnp.concatenate([s0, s1, s2], axis=0)


# speedup vs baseline: 12.1778x; 1.2761x over previous
import functools
import math

import jax
import jax.numpy as jnp
import numpy as np
from jax.experimental import pallas as pl
from jax.experimental.pallas import tpu as pltpu

B = 64
PTS = 256
F0 = 64
E = 128
H = 4
DH = E // H
NODES = [16, 4]
NCOND = 6
SLOPE = 0.01
GPP = 16


def _lrelu(x):
    return jnp.where(x >= 0, x, SLOPE * x)


def _prep_ffn(p):
    return {'W1': p['W1'], 'b1': p['b1'][None, :],
            'W2': p['W2'], 'b2': p['b2'][None, :]}


def _prep_cnu(p):
    return {'emb': _prep_ffn(p['emb']), 'glob': _prep_ffn(p['glob']),
            'out': _prep_ffn(p['out'])}


def _prep_disc(p):
    return {'layers': [_prep_cnu(lp) for lp in p['layers']],
            'disc': _prep_ffn(p['disc'])}


def _prep_pool(p):
    return {'xcent': p['xcent_base'],
            'Wq': p['Wq'], 'bq': p['bq'][None, :],
            'Wk': p['Wk'], 'bk': p['bk'][None, :],
            'Wv': p['Wv'], 'bv': p['bv'][None, :],
            'Wo': p['Wo'], 'bo': p['bo'][None, :]}


def _mm(a, b):
    return jax.lax.dot_general(a, b, (((a.ndim - 1,), (0,)), ((), ())),
                               preferred_element_type=jnp.float32)


def _ffn(w, x, final_linear=False):
    h = _lrelu(_mm(x, w['W1']) + w['b1'])
    o = _mm(h, w['W2']) + w['b2']
    return o if final_linear else _lrelu(o)


def _gmp(x2d, r):
    d = x2d.shape[-1]
    x3 = x2d.reshape(GPP, r, d)
    mean = jnp.mean(x3, axis=1)
    mad = jnp.mean(jnp.abs(x3 - mean[:, None, :]), axis=1)
    mx = jnp.max(x3, axis=1)
    cnt = jnp.full((GPP, 1), float(r), jnp.float32)
    return cnt, mean, mad, mx


def _cnu(w, x2d, r):
    xl = _ffn(w['emb'], x2d)
    cnt, mean, mad, mx = _gmp(xl, r)
    g = jnp.concatenate([cnt, mean, mad, mx], axis=-1)
    xg = _ffn(w['glob'], g)
    n_glob = xg.shape[-1]
    xgb = jnp.broadcast_to(xg[:, None, :],
                           (GPP, r, n_glob)).reshape(GPP * r, n_glob)
    cat = jnp.concatenate([xl, xgb], axis=-1)
    return _ffn(w['out'], cat, final_linear=True)


def _disc(w, x2d, cond, r):
    for lw in w['layers']:
        x2d = x2d + _cnu(lw, x2d, r)
    cnt, mean, mad, mx = _gmp(x2d, r)
    inp = jnp.concatenate([cnt, mean, mad, mx, cond], axis=-1)
    return _ffn(w['disc'], inp, final_linear=True)


def _pool(w, x2d, s, r):
    scale = math.sqrt(DH)
    q = _mm(w['xcent'], w['Wq']) + w['bq']
    k = _mm(x2d, w['Wk']) + w['bk']
    v = _mm(x2d, w['Wv']) + w['bv']
    heads = []
    for h in range(H):
        sl = slice(h * DH, (h + 1) * DH)
        sc = jax.lax.dot_general(k[:, sl], q[:, sl], (((1,), (1,)), ((), ())),
                                 preferred_element_type=jnp.float32)
        sc = (sc / scale).reshape(GPP, s, r)
        m = jnp.max(sc, axis=1, keepdims=True)
        e = jnp.exp(sc - m)
        a = e / jnp.sum(e, axis=1, keepdims=True)
        v3 = v[:, sl].reshape(GPP, s, DH)
        heads.append(jax.lax.dot_general(
            a, v3, (((1,), (1,)), ((0,), (0,))),
            preferred_element_type=jnp.float32))
    o = jnp.concatenate(heads, axis=-1).reshape(GPP * r, E)
    return _mm(o, w['Wo']) + w['bo']


def _body(x_ref, cond_ref, p_ref, o0_ref, o1_ref, o2_ref):
    x = x_ref[...]
    cond = cond_ref[...]
    p = jax.tree.map(lambda ref: ref[...], p_ref,
                     is_leaf=lambda n: hasattr(n, 'dtype') and hasattr(n, 'at'))
    w = {
        'disc': [_prep_disc(q) for q in p['disc']],
        'emb': [{'inp': _prep_ffn(q['inp']), 'cnu': _prep_cnu(q['cnu'])}
                for q in p['emb']],
        'pool': [_prep_pool(q) for q in p['pool']],
    }

    o0_ref[...] = _disc(w['disc'][0], x, cond, PTS)

    x0 = _ffn(w['emb'][0]['inp'], x, final_linear=True)
    xe = _cnu(w['emb'][0]['cnu'], x0, PTS) + x0
    x1 = _pool(w['pool'][0], xe, PTS, NODES[0])

    o1_ref[...] = _disc(w['disc'][1], x1, cond, NODES[0])

    xi = _ffn(w['emb'][1]['inp'], x1, final_linear=True)
    xe1 = _cnu(w['emb'][1]['cnu'], xi, NODES[0]) + xi
    x2 = _pool(w['pool'][1], xe1, NODES[0], NODES[1])

    o2_ref[...] = _disc(w['disc'][2], x2, cond, NODES[1])


def _full_spec(a):
    nd = a.ndim
    return pl.BlockSpec(a.shape, lambda i, _n=nd: (0,) * _n)


@jax.jit
def kernel(x, batchidx, condition, params):
    del batchidx
    grid = B // GPP
    wspecs = jax.tree.map(_full_spec, params)
    s0, s1, s2 = pl.pallas_call(
        _body,
        grid=(grid,),
        in_specs=[
            pl.BlockSpec((GPP * PTS, F0), lambda i: (i, 0)),
            pl.BlockSpec((GPP, NCOND), lambda i: (i, 0)),
            wspecs,
        ],
        out_specs=[pl.BlockSpec((GPP, 1), lambda i: (i, 0))] * 3,
        out_shape=[jax.ShapeDtypeStruct((B, 1), jnp.float32)] * 3,
        compiler_params=pltpu.CompilerParams(
            dimension_semantics=("arbitrary",)),
    )(x, condition, params)
    return jnp.concatenate([s0, s1, s2], axis=0)


# speedup vs baseline: 12.3263x; 1.0122x over previous
import functools
import math

import jax
import jax.numpy as jnp
import numpy as np
from jax.experimental import pallas as pl
from jax.experimental.pallas import tpu as pltpu

B = 64
PTS = 256
F0 = 64
E = 128
H = 4
DH = E // H
NODES = [16, 4]
NCOND = 6
SLOPE = 0.01
GPP = 16


def _lrelu(x):
    return jnp.maximum(x, SLOPE * x)


def _prep_ffn(p):
    return {'W1': p['W1'], 'b1': p['b1'][None, :],
            'W2': p['W2'], 'b2': p['b2'][None, :]}


def _prep_cnu(p):
    return {'emb': _prep_ffn(p['emb']), 'glob': _prep_ffn(p['glob']),
            'out': _prep_ffn(p['out'])}


def _prep_disc(p):
    return {'layers': [_prep_cnu(lp) for lp in p['layers']],
            'disc': _prep_ffn(p['disc'])}


def _prep_pool(p):
    return {'xcent': p['xcent_base'],
            'Wq': p['Wq'], 'bq': p['bq'][None, :],
            'Wk': p['Wk'], 'bk': p['bk'][None, :],
            'Wv': p['Wv'], 'bv': p['bv'][None, :],
            'Wo': p['Wo'], 'bo': p['bo'][None, :]}


def _mm(a, b):
    return jax.lax.dot_general(a, b, (((a.ndim - 1,), (0,)), ((), ())),
                               preferred_element_type=jnp.float32)


def _ffn(w, x, final_linear=False):
    h = _lrelu(_mm(x, w['W1']) + w['b1'])
    o = _mm(h, w['W2']) + w['b2']
    return o if final_linear else _lrelu(o)


def _gmp(x2d, r):
    d = x2d.shape[-1]
    x3 = x2d.reshape(GPP, r, d)
    mean = jnp.mean(x3, axis=1)
    mad = jnp.mean(jnp.abs(x3 - mean[:, None, :]), axis=1)
    mx = jnp.max(x3, axis=1)
    cnt = jnp.full((GPP, 1), float(r), jnp.float32)
    return cnt, mean, mad, mx


def _cnu(w, x2d, r):
    xl = _ffn(w['emb'], x2d)
    cnt, mean, mad, mx = _gmp(xl, r)
    g = jnp.concatenate([cnt, mean, mad, mx], axis=-1)
    xg = _ffn(w['glob'], g)
    n_glob = xg.shape[-1]
    xgb = jnp.broadcast_to(xg[:, None, :],
                           (GPP, r, n_glob)).reshape(GPP * r, n_glob)
    cat = jnp.concatenate([xl, xgb], axis=-1)
    return _ffn(w['out'], cat, final_linear=True)


def _disc(w, x2d, cond, r):
    for lw in w['layers']:
        x2d = x2d + _cnu(lw, x2d, r)
    cnt, mean, mad, mx = _gmp(x2d, r)
    inp = jnp.concatenate([cnt, mean, mad, mx, cond], axis=-1)
    return _ffn(w['disc'], inp, final_linear=True)


def _pool(w, x2d, s, r):
    scale = math.sqrt(DH)
    q = _mm(w['xcent'], w['Wq']) + w['bq']
    k = _mm(x2d, w['Wk']) + w['bk']
    v = _mm(x2d, w['Wv']) + w['bv']
    heads = []
    for h in range(H):
        sl = slice(h * DH, (h + 1) * DH)
        sc = jax.lax.dot_general(k[:, sl], q[:, sl], (((1,), (1,)), ((), ())),
                                 preferred_element_type=jnp.float32)
        sc = (sc / scale).reshape(GPP, s, r)
        m = jnp.max(sc, axis=1, keepdims=True)
        e = jnp.exp(sc - m)
        a = e / jnp.sum(e, axis=1, keepdims=True)
        v3 = v[:, sl].reshape(GPP, s, DH)
        heads.append(jax.lax.dot_general(
            a, v3, (((1,), (1,)), ((0,), (0,))),
            preferred_element_type=jnp.float32))
    o = jnp.concatenate(heads, axis=-1).reshape(GPP * r, E)
    return _mm(o, w['Wo']) + w['bo']


def _body(x_ref, cond_ref, p_ref, o0_ref, o1_ref, o2_ref):
    x = x_ref[...]
    cond = cond_ref[...]
    p = jax.tree.map(lambda ref: ref[...], p_ref,
                     is_leaf=lambda n: hasattr(n, 'dtype') and hasattr(n, 'at'))
    w = {
        'disc': [_prep_disc(q) for q in p['disc']],
        'emb': [{'inp': _prep_ffn(q['inp']), 'cnu': _prep_cnu(q['cnu'])}
                for q in p['emb']],
        'pool': [_prep_pool(q) for q in p['pool']],
    }

    o0_ref[...] = _disc(w['disc'][0], x, cond, PTS)

    x0 = _ffn(w['emb'][0]['inp'], x, final_linear=True)
    xe = _cnu(w['emb'][0]['cnu'], x0, PTS) + x0
    x1 = _pool(w['pool'][0], xe, PTS, NODES[0])

    o1_ref[...] = _disc(w['disc'][1], x1, cond, NODES[0])

    xi = _ffn(w['emb'][1]['inp'], x1, final_linear=True)
    xe1 = _cnu(w['emb'][1]['cnu'], xi, NODES[0]) + xi
    x2 = _pool(w['pool'][1], xe1, NODES[0], NODES[1])

    o2_ref[...] = _disc(w['disc'][2], x2, cond, NODES[1])


def _full_spec(a):
    nd = a.ndim
    return pl.BlockSpec(a.shape, lambda i, _n=nd: (0,) * _n)


@jax.jit
def kernel(x, batchidx, condition, params):
    del batchidx
    grid = B // GPP
    wspecs = jax.tree.map(_full_spec, params)
    s0, s1, s2 = pl.pallas_call(
        _body,
        grid=(grid,),
        in_specs=[
            pl.BlockSpec((GPP * PTS, F0), lambda i: (i, 0)),
            pl.BlockSpec((GPP, NCOND), lambda i: (i, 0)),
            wspecs,
        ],
        out_specs=[pl.BlockSpec((GPP, 1), lambda i: (i, 0))] * 3,
        out_shape=[jax.ShapeDtypeStruct((B, 1), jnp.float32)] * 3,
        compiler_params=pltpu.CompilerParams(
            dimension_semantics=("parallel",)),
    )(x, condition, params)
    return jnp.concatenate([s0, s1, s2], axis=0)
